# baseline (speedup 1.0000x reference)
import functools

import jax
import jax.numpy as jnp
from jax import lax
from jax.experimental import pallas as pl
from jax.experimental.pallas import tpu as pltpu
from jax.experimental.pallas import tpu_sc as plsc

N = 10000
E = 160000
H = 128
NPAD = 10240
NSUB = 16
NODES_PER_SUB = NPAD // NSUB


def _sc_first_edge(edge_index, ef_flat):
    mesh = plsc.VectorSubcoreMesh(core_axis_name="c", subcore_axis_name="s")

    @functools.partial(
        pl.kernel,
        mesh=mesh,
        compiler_params=pltpu.CompilerParams(needs_layout_passes=False,
                                             use_tc_tiling_on_sc=False),
        out_type=jax.ShapeDtypeStruct((3 * NPAD,), jnp.int32),
        scratch_types=[
            pltpu.VMEM((E // NSUB,), jnp.int32),
            pltpu.VMEM((NPAD,), jnp.int32),
            pltpu.VMEM_SHARED((NSUB, NPAD), jnp.int32),
            pltpu.VMEM((NSUB, NODES_PER_SUB), jnp.int32),
            pltpu.VMEM((NODES_PER_SUB,), jnp.int32),
            pltpu.VMEM((NODES_PER_SUB,), jnp.int32),
            pltpu.VMEM((NODES_PER_SUB,), jnp.int32),
            pltpu.VMEM((NODES_PER_SUB,), jnp.int32),
            pltpu.VMEM((NODES_PER_SUB,), jnp.int32),
            pltpu.SemaphoreType.DMA,
        ],
    )
    def k(ei_hbm, ef_hbm, packed_out, dst_v, local_v, shared, mbuf, fe_v,
          sidx_v, sidx2_v, r0_v, r1_v, sem):
        c = lax.axis_index("c")
        s = lax.axis_index("s")

        def run(edge_base, eps):
            pltpu.sync_copy(ei_hbm.at[1, pl.ds(edge_base + s * eps, eps)],
                            dst_v.at[pl.ds(0, eps)])

            def init_body(i, _):
                local_v[pl.ds(i * 16, 16)] = jnp.full((16,), E, jnp.int32)
                return _
            lax.fori_loop(0, NPAD // 16, init_body, None)

            iota16 = lax.iota(jnp.int32, 16)

            def scanA(j, _):
                d = dst_v[pl.ds(j * 16, 16)]
                eid = edge_base + s * eps + j * 16 + iota16
                cur = plsc.load_gather(local_v, [d])
                plsc.store_scatter(local_v, [d], jnp.minimum(cur, eid))
                return _
            lax.fori_loop(0, eps // 16, scanA, None)

            def cond(act):
                return plsc.all_reduce_population_count(act)[0] > 0

            def scanB(j, _):
                d = dst_v[pl.ds(j * 16, 16)]
                eid = edge_base + s * eps + j * 16 + iota16

                def w_body(act):
                    plsc.store_scatter(local_v, [d], eid, mask=act)
                    return eid < plsc.load_gather(local_v, [d])

                act0 = eid < plsc.load_gather(local_v, [d])
                lax.while_loop(cond, w_body, act0)
                return _
            lax.fori_loop(0, eps // 16, scanB, None)

            pltpu.sync_copy(local_v, shared.at[s])
            plsc.subcore_barrier()
            for r in range(NSUB):
                pltpu.sync_copy(
                    shared.at[r, pl.ds(s * NODES_PER_SUB, NODES_PER_SUB)],
                    mbuf.at[r])

            def merge_body(j, _):
                v = mbuf[0, pl.ds(j * 16, 16)]
                for r in range(1, NSUB):
                    v = jnp.minimum(v, mbuf[r, pl.ds(j * 16, 16)])
                fe_v[pl.ds(j * 16, 16)] = v
                sv = jnp.minimum(v, E - 1) * 2
                sidx_v[pl.ds(j * 16, 16)] = sv
                sidx2_v[pl.ds(j * 16, 16)] = sv + 1
                return _
            lax.fori_loop(0, NODES_PER_SUB // 16, merge_body, None)

            pltpu.async_copy(ef_hbm.at[sidx_v], r0_v, sem).wait()
            pltpu.async_copy(ef_hbm.at[sidx2_v], r1_v, sem).wait()

            base = s * NODES_PER_SUB
            pltpu.sync_copy(fe_v,
                            packed_out.at[pl.ds(base, NODES_PER_SUB)])
            pltpu.sync_copy(r0_v,
                            packed_out.at[pl.ds(NPAD + base, NODES_PER_SUB)])
            pltpu.sync_copy(r1_v,
                            packed_out.at[pl.ds(2 * NPAD + base,
                                                NODES_PER_SUB)])

        @pl.when(c == 0)
        def _():
            run(0, E // NSUB)

    return k(edge_index, ef_flat)


def _dot_t(x, w):
    return lax.dot_general(x, w, (((1,), (1,)), ((), ())),
                           preferred_element_type=jnp.float32)


def _tc_dense(nf_ref, fe_ref, e0_ref, e1_ref, wn_ref, bn1_ref, bn2_ref,
              we_ref, be1_ref, be2_ref, wnm_ref, bnm_ref, wfc_ref, bfc_ref,
              o_ref):
    def gates_to_h(g):
        i = jax.nn.sigmoid(g[:, 0 * H:1 * H])
        gg = jnp.tanh(g[:, 2 * H:3 * H])
        o = jax.nn.sigmoid(g[:, 3 * H:4 * H])
        return o * jnp.tanh(i * gg)

    fe = fe_ref[0]
    e0 = lax.bitcast_convert_type(e0_ref[0], jnp.float32)
    e1 = lax.bitcast_convert_type(e1_ref[0], jnp.float32)

    bn = bn1_ref[...] + bn2_ref[...]
    be = be1_ref[...] + be2_ref[...]
    wnm = wnm_ref[...]
    we = we_ref[...]
    h1 = gates_to_h(_dot_t(nf_ref[...], wn_ref[...]) + bn)
    ge = _dot_t(e0, we[:, 0:1]) + _dot_t(e1, we[:, 1:2]) + be
    eh = gates_to_h(ge)
    t = _dot_t(eh, wnm[:, :H]) + _dot_t(h1, wnm[:, H:])
    has_in = fe < E
    m = jax.nn.sigmoid(jnp.where(has_in, t, 0.0) + bnm_ref[...])
    h2 = gates_to_h(_dot_t(m, wn_ref[...]) + bn)
    lg = _dot_t(h2, wfc_ref[...]) + bfc_ref[...]
    mx = jnp.max(lg, axis=1, keepdims=True)
    lse = mx + jnp.log(jnp.sum(jnp.exp(lg - mx), axis=1, keepdims=True))
    o_ref[...] = lg - lse


def kernel(node_feat, edge_feat, Wih_n, Whh_n, bih_n, bhh_n, Wih_e, Whh_e,
           bih_e, bhh_e, W_nm, b_nm, W_em, b_em, W_fc, b_fc, edge_index):
    ef_bits = lax.bitcast_convert_type(edge_feat.reshape(2 * E), jnp.int32)
    packed = _sc_first_edge(edge_index, ef_bits)
    pk3 = packed.reshape(3, NPAD, 1)

    nf = node_feat.reshape(N, 21)

    R = 1000
    grid = N // R
    seg = lambda r: pl.BlockSpec((1, R, 1), lambda i, _r=r: (_r, i, 0))
    full = lambda shape: pl.BlockSpec(shape, lambda i: (0, 0))
    out = pl.pallas_call(
        _tc_dense,
        grid=(grid,),
        in_specs=[
            pl.BlockSpec((R, 21), lambda i: (i, 0)),
            seg(0), seg(1), seg(2),
            full((4 * H, 21)),
            full((1, 4 * H)),
            full((1, 4 * H)),
            full((4 * H, 2)),
            full((1, 4 * H)),
            full((1, 4 * H)),
            full((21, 2 * H)),
            full((1, 21)),
            full((4, H)),
            full((1, 4)),
        ],
        out_specs=pl.BlockSpec((R, 4), lambda i: (i, 0)),
        out_shape=jax.ShapeDtypeStruct((N, 4), jnp.float32),
    )(
        nf,
        pk3, pk3, pk3,
        Wih_n,
        bih_n.reshape(1, 4 * H),
        bhh_n.reshape(1, 4 * H),
        Wih_e,
        bih_e.reshape(1, 4 * H),
        bhh_e.reshape(1, 4 * H),
        W_nm,
        b_nm.reshape(1, 21),
        W_fc,
        b_fc.reshape(1, 4),
    )
    return out


# speedup vs baseline: 1.8655x; 1.8655x over previous
import functools

import jax
import jax.numpy as jnp
from jax import lax
from jax.experimental import pallas as pl
from jax.experimental.pallas import tpu as pltpu
from jax.experimental.pallas import tpu_sc as plsc

N = 10000
E = 160000
H = 128
NPAD = 10240
NSUB = 16
EDGES_PER_SUB = E // NSUB
VECS_PER_SUB = EDGES_PER_SUB // 16
NODES_PER_SUB = NPAD // NSUB


def _sc_first_edge(edge_index, ef0, ef1):
    mesh = plsc.VectorSubcoreMesh(core_axis_name="c", subcore_axis_name="s")

    @functools.partial(
        pl.kernel,
        mesh=mesh,
        compiler_params=pltpu.CompilerParams(needs_layout_passes=False,
                                             use_tc_tiling_on_sc=False),
        out_type=(
            jax.ShapeDtypeStruct((NPAD,), jnp.int32),
            jax.ShapeDtypeStruct((NPAD,), jnp.float32),
            jax.ShapeDtypeStruct((NPAD,), jnp.float32),
        ),
        scratch_types=[
            pltpu.VMEM((EDGES_PER_SUB,), jnp.int32),
            pltpu.VMEM((NPAD,), jnp.int32),
            pltpu.VMEM_SHARED((NSUB, NPAD), jnp.int32),
            pltpu.VMEM((NSUB, NODES_PER_SUB), jnp.int32),
            pltpu.VMEM((NODES_PER_SUB,), jnp.int32),
            pltpu.VMEM((NODES_PER_SUB,), jnp.int32),
            pltpu.VMEM((NODES_PER_SUB,), jnp.float32),
            pltpu.VMEM((NODES_PER_SUB,), jnp.float32),
            pltpu.SemaphoreType.DMA,
        ],
    )
    def k(ei_hbm, ef0_hbm, ef1_hbm, fe_out, e0_out, e1_out, dst_v, local_v,
          shared, mbuf, fe_v, sidx_v, r0_v, r1_v, sem):
        c = lax.axis_index("c")
        s = lax.axis_index("s")

        @pl.when(c == 0)
        def _():
            pltpu.sync_copy(
                ei_hbm.at[1, pl.ds(s * EDGES_PER_SUB, EDGES_PER_SUB)], dst_v)

            def init_body(i, _):
                local_v[pl.ds(i * 16, 16)] = jnp.full((16,), E, jnp.int32)
                return _
            lax.fori_loop(0, NPAD // 16, init_body, None)

            iota16 = lax.iota(jnp.int32, 16)

            def scanA(j, _):
                d = dst_v[pl.ds(j * 16, 16)]
                eid = s * EDGES_PER_SUB + j * 16 + iota16
                cur = plsc.load_gather(local_v, [d])
                plsc.store_scatter(local_v, [d], jnp.minimum(cur, eid))
                return _
            lax.fori_loop(0, VECS_PER_SUB, scanA, None)

            def cond(act):
                return plsc.all_reduce_population_count(act)[0] > 0

            def scanB(j, _):
                d = dst_v[pl.ds(j * 16, 16)]
                eid = s * EDGES_PER_SUB + j * 16 + iota16

                def w_body(act):
                    plsc.store_scatter(local_v, [d], eid, mask=act)
                    return eid < plsc.load_gather(local_v, [d])

                act0 = eid < plsc.load_gather(local_v, [d])
                lax.while_loop(cond, w_body, act0)
                return _
            lax.fori_loop(0, VECS_PER_SUB, scanB, None)

            pltpu.sync_copy(local_v, shared.at[s])
            plsc.subcore_barrier()
            for r in range(NSUB):
                pltpu.sync_copy(
                    shared.at[r, pl.ds(s * NODES_PER_SUB, NODES_PER_SUB)],
                    mbuf.at[r])

            def merge_body(j, _):
                v = mbuf[0, pl.ds(j * 16, 16)]
                for r in range(1, NSUB):
                    v = jnp.minimum(v, mbuf[r, pl.ds(j * 16, 16)])
                fe_v[pl.ds(j * 16, 16)] = v
                sidx_v[pl.ds(j * 16, 16)] = jnp.minimum(v, E - 1)
                return _
            lax.fori_loop(0, NODES_PER_SUB // 16, merge_body, None)

            pltpu.async_copy(ef0_hbm.at[sidx_v], r0_v, sem).wait()
            pltpu.async_copy(ef1_hbm.at[sidx_v], r1_v, sem).wait()

            base = s * NODES_PER_SUB
            pltpu.sync_copy(fe_v, fe_out.at[pl.ds(base, NODES_PER_SUB)])
            pltpu.sync_copy(r0_v, e0_out.at[pl.ds(base, NODES_PER_SUB)])
            pltpu.sync_copy(r1_v, e1_out.at[pl.ds(base, NODES_PER_SUB)])

    return k(edge_index, ef0, ef1)


def _dot_t(x, w):
    return lax.dot_general(x, w, (((1,), (1,)), ((), ())),
                           preferred_element_type=jnp.float32)


def _tc_dense(nf_ref, e0_ref, e1_ref, fe_ref, wn_ref, bn1_ref, bn2_ref,
              we_ref, be1_ref, be2_ref, wnm_ref, bnm_ref, wfc_ref, bfc_ref,
              o_ref):
    def gates_to_h(g):
        i = jax.nn.sigmoid(g[:, 0 * H:1 * H])
        gg = jnp.tanh(g[:, 2 * H:3 * H])
        o = jax.nn.sigmoid(g[:, 3 * H:4 * H])
        return o * jnp.tanh(i * gg)

    bn = bn1_ref[...] + bn2_ref[...]
    be = be1_ref[...] + be2_ref[...]
    wnm = wnm_ref[...]
    we = we_ref[...]
    h1 = gates_to_h(_dot_t(nf_ref[...], wn_ref[...]) + bn)
    ge = (_dot_t(e0_ref[...], we[:, 0:1]) + _dot_t(e1_ref[...], we[:, 1:2])
          + be)
    eh = gates_to_h(ge)
    t = _dot_t(eh, wnm[:, :H]) + _dot_t(h1, wnm[:, H:])
    has_in = fe_ref[...] < E
    m = jax.nn.sigmoid(jnp.where(has_in, t, 0.0) + bnm_ref[...])
    h2 = gates_to_h(_dot_t(m, wn_ref[...]) + bn)
    lg = _dot_t(h2, wfc_ref[...]) + bfc_ref[...]
    mx = jnp.max(lg, axis=1, keepdims=True)
    lse = mx + jnp.log(jnp.sum(jnp.exp(lg - mx), axis=1, keepdims=True))
    o_ref[...] = lg - lse


def kernel(node_feat, edge_feat, Wih_n, Whh_n, bih_n, bhh_n, Wih_e, Whh_e,
           bih_e, bhh_e, W_nm, b_nm, W_em, b_em, W_fc, b_fc, edge_index):
    ef0 = edge_feat[:, 0, 0]
    ef1 = edge_feat[:, 0, 1]
    fe, e0, e1 = _sc_first_edge(edge_index, ef0, ef1)

    nf = node_feat.reshape(N, 21)

    R = 1000
    grid = N // R
    full = lambda shape: pl.BlockSpec(shape, lambda i: (0, 0))
    out = pl.pallas_call(
        _tc_dense,
        grid=(grid,),
        in_specs=[
            pl.BlockSpec((R, 21), lambda i: (i, 0)),
            pl.BlockSpec((R, 1), lambda i: (i, 0)),
            pl.BlockSpec((R, 1), lambda i: (i, 0)),
            pl.BlockSpec((R, 1), lambda i: (i, 0)),
            full((4 * H, 21)),
            full((1, 4 * H)),
            full((1, 4 * H)),
            full((4 * H, 2)),
            full((1, 4 * H)),
            full((1, 4 * H)),
            full((21, 2 * H)),
            full((1, 21)),
            full((4, H)),
            full((1, 4)),
        ],
        out_specs=pl.BlockSpec((R, 4), lambda i: (i, 0)),
        out_shape=jax.ShapeDtypeStruct((N, 4), jnp.float32),
    )(
        nf,
        e0.reshape(NPAD, 1),
        e1.reshape(NPAD, 1),
        fe.reshape(NPAD, 1),
        Wih_n,
        bih_n.reshape(1, 4 * H),
        bhh_n.reshape(1, 4 * H),
        Wih_e,
        bih_e.reshape(1, 4 * H),
        bhh_e.reshape(1, 4 * H),
        W_nm,
        b_nm.reshape(1, 21),
        W_fc,
        b_fc.reshape(1, 4),
    )
    return out


# speedup vs baseline: 1.9773x; 1.0599x over previous
import functools

import jax
import jax.numpy as jnp
from jax import lax
from jax.experimental import pallas as pl
from jax.experimental.pallas import tpu as pltpu
from jax.experimental.pallas import tpu_sc as plsc

N = 10000
E = 160000
H = 128
NPAD = 10240
NSUB = 16
EDGES_PER_SUB = E // NSUB
VECS_PER_SUB = EDGES_PER_SUB // 16
NODES_PER_SUB = NPAD // NSUB


def _sc_first_edge(edge_index, ef0, ef1):
    mesh = plsc.VectorSubcoreMesh(core_axis_name="c", subcore_axis_name="s")

    @functools.partial(
        pl.kernel,
        mesh=mesh,
        compiler_params=pltpu.CompilerParams(needs_layout_passes=False,
                                             use_tc_tiling_on_sc=False),
        out_type=(
            jax.ShapeDtypeStruct((NPAD,), jnp.int32),
            jax.ShapeDtypeStruct((NPAD,), jnp.float32),
            jax.ShapeDtypeStruct((NPAD,), jnp.float32),
        ),
        scratch_types=[
            pltpu.VMEM((EDGES_PER_SUB,), jnp.int32),
            pltpu.VMEM((NPAD,), jnp.int32),
            pltpu.VMEM_SHARED((NSUB, NPAD), jnp.int32),
            pltpu.VMEM((NSUB, NODES_PER_SUB), jnp.int32),
            pltpu.VMEM((NODES_PER_SUB,), jnp.int32),
            pltpu.VMEM((NODES_PER_SUB,), jnp.int32),
            pltpu.VMEM((NODES_PER_SUB,), jnp.float32),
            pltpu.VMEM((NODES_PER_SUB,), jnp.float32),
            pltpu.SemaphoreType.DMA,
        ],
    )
    def k(ei_hbm, ef0_hbm, ef1_hbm, fe_out, e0_out, e1_out, dst_v, local_v,
          shared, mbuf, fe_v, sidx_v, r0_v, r1_v, sem):
        c = lax.axis_index("c")
        s = lax.axis_index("s")

        @pl.when(c == 0)
        def _():
            pltpu.sync_copy(
                ei_hbm.at[1, pl.ds(s * EDGES_PER_SUB, EDGES_PER_SUB)], dst_v)

            def init_body(i, _):
                local_v[pl.ds(i * 16, 16)] = jnp.full((16,), E, jnp.int32)
                return _
            lax.fori_loop(0, NPAD // 16, init_body, None)

            iota16 = lax.iota(jnp.int32, 16)

            def cond(act):
                return plsc.all_reduce_population_count(act)[0] > 0

            def scan(j, _):
                d = dst_v[pl.ds(j * 16, 16)]
                eid = s * EDGES_PER_SUB + j * 16 + iota16
                cur = plsc.load_gather(local_v, [d])
                plsc.store_scatter(local_v, [d], jnp.minimum(cur, eid))

                def w_body(act):
                    plsc.store_scatter(local_v, [d], eid, mask=act)
                    return eid < plsc.load_gather(local_v, [d])

                act0 = eid < plsc.load_gather(local_v, [d])
                lax.while_loop(cond, w_body, act0)
                return _
            lax.fori_loop(0, VECS_PER_SUB, scan, None)

            pltpu.sync_copy(local_v, shared.at[s])
            plsc.subcore_barrier()
            for r in range(NSUB):
                pltpu.sync_copy(
                    shared.at[r, pl.ds(s * NODES_PER_SUB, NODES_PER_SUB)],
                    mbuf.at[r])

            def merge_body(j, _):
                v = mbuf[0, pl.ds(j * 16, 16)]
                for r in range(1, NSUB):
                    v = jnp.minimum(v, mbuf[r, pl.ds(j * 16, 16)])
                fe_v[pl.ds(j * 16, 16)] = v
                sidx_v[pl.ds(j * 16, 16)] = jnp.minimum(v, E - 1)
                return _
            lax.fori_loop(0, NODES_PER_SUB // 16, merge_body, None)

            pltpu.async_copy(ef0_hbm.at[sidx_v], r0_v, sem).wait()
            pltpu.async_copy(ef1_hbm.at[sidx_v], r1_v, sem).wait()

            base = s * NODES_PER_SUB
            pltpu.sync_copy(fe_v, fe_out.at[pl.ds(base, NODES_PER_SUB)])
            pltpu.sync_copy(r0_v, e0_out.at[pl.ds(base, NODES_PER_SUB)])
            pltpu.sync_copy(r1_v, e1_out.at[pl.ds(base, NODES_PER_SUB)])

    return k(edge_index, ef0, ef1)


def _dot_t(x, w):
    return lax.dot_general(x, w, (((1,), (1,)), ((), ())),
                           preferred_element_type=jnp.float32)


def _tc_dense(nf_ref, e0_ref, e1_ref, fe_ref, wn_ref, bn1_ref, bn2_ref,
              we_ref, be1_ref, be2_ref, wnm_ref, bnm_ref, wfc_ref, bfc_ref,
              o_ref):
    def gates_to_h(g):
        i = jax.nn.sigmoid(g[:, 0 * H:1 * H])
        gg = jnp.tanh(g[:, 2 * H:3 * H])
        o = jax.nn.sigmoid(g[:, 3 * H:4 * H])
        return o * jnp.tanh(i * gg)

    bn = bn1_ref[...] + bn2_ref[...]
    be = be1_ref[...] + be2_ref[...]
    wnm = wnm_ref[...]
    we = we_ref[...]
    h1 = gates_to_h(_dot_t(nf_ref[...], wn_ref[...]) + bn)
    ge = (_dot_t(e0_ref[...], we[:, 0:1]) + _dot_t(e1_ref[...], we[:, 1:2])
          + be)
    eh = gates_to_h(ge)
    t = _dot_t(eh, wnm[:, :H]) + _dot_t(h1, wnm[:, H:])
    has_in = fe_ref[...] < E
    m = jax.nn.sigmoid(jnp.where(has_in, t, 0.0) + bnm_ref[...])
    h2 = gates_to_h(_dot_t(m, wn_ref[...]) + bn)
    lg = _dot_t(h2, wfc_ref[...]) + bfc_ref[...]
    mx = jnp.max(lg, axis=1, keepdims=True)
    lse = mx + jnp.log(jnp.sum(jnp.exp(lg - mx), axis=1, keepdims=True))
    o_ref[...] = lg - lse


def kernel(node_feat, edge_feat, Wih_n, Whh_n, bih_n, bhh_n, Wih_e, Whh_e,
           bih_e, bhh_e, W_nm, b_nm, W_em, b_em, W_fc, b_fc, edge_index):
    ef0 = edge_feat[:, 0, 0]
    ef1 = edge_feat[:, 0, 1]
    fe, e0, e1 = _sc_first_edge(edge_index, ef0, ef1)

    nf = node_feat.reshape(N, 21)

    R = 2000
    grid = N // R
    full = lambda shape: pl.BlockSpec(shape, lambda i: (0, 0))
    out = pl.pallas_call(
        _tc_dense,
        grid=(grid,),
        in_specs=[
            pl.BlockSpec((R, 21), lambda i: (i, 0)),
            pl.BlockSpec((R, 1), lambda i: (i, 0)),
            pl.BlockSpec((R, 1), lambda i: (i, 0)),
            pl.BlockSpec((R, 1), lambda i: (i, 0)),
            full((4 * H, 21)),
            full((1, 4 * H)),
            full((1, 4 * H)),
            full((4 * H, 2)),
            full((1, 4 * H)),
            full((1, 4 * H)),
            full((21, 2 * H)),
            full((1, 21)),
            full((4, H)),
            full((1, 4)),
        ],
        out_specs=pl.BlockSpec((R, 4), lambda i: (i, 0)),
        out_shape=jax.ShapeDtypeStruct((N, 4), jnp.float32),
    )(
        nf,
        e0.reshape(NPAD, 1),
        e1.reshape(NPAD, 1),
        fe.reshape(NPAD, 1),
        Wih_n,
        bih_n.reshape(1, 4 * H),
        bhh_n.reshape(1, 4 * H),
        Wih_e,
        bih_e.reshape(1, 4 * H),
        bhh_e.reshape(1, 4 * H),
        W_nm,
        b_nm.reshape(1, 21),
        W_fc,
        b_fc.reshape(1, 4),
    )
    return out
